# baseline (speedup 1.0000x reference)
import functools

import jax
import jax.numpy as jnp
from jax import lax
from jax.experimental import pallas as pl
from jax.experimental.pallas import tpu as pltpu
from jax.experimental.pallas import tpu_sc as plsc

_NC = 2
_NSC = 16
_NW = _NC * _NSC

_K = 2
_BLK = 1024


def _silu(v):
    return v / (1.0 + jnp.exp(-v))


def _router_body(x_ref, wgate_ref, p0_ref, p1_ref, w0_ref, w1_ref,
                 counts_ref, loss_ref):
    x = x_ref[...]
    wg = wgate_ref[...]
    T = x.shape[0]
    E = wg.shape[0]
    logits = lax.dot_general(x, wg, (((1,), (1,)), ((), ())),
                             preferred_element_type=jnp.float32)
    m = jnp.max(logits, axis=-1, keepdims=True)
    ex = jnp.exp(logits - m)
    probs = ex / jnp.sum(ex, axis=-1, keepdims=True)
    eids = lax.broadcasted_iota(jnp.int32, probs.shape, 1)
    v1 = jnp.max(probs, axis=-1, keepdims=True)
    i1 = jnp.min(jnp.where(probs == v1, eids, E), axis=-1, keepdims=True)
    sel1 = eids == i1
    p2 = jnp.where(sel1, -1.0, probs)
    v2 = jnp.max(p2, axis=-1, keepdims=True)
    i2 = jnp.min(jnp.where(p2 == v2, eids, E), axis=-1, keepdims=True)
    sel2 = eids == i2
    denom = v1 + v2
    w0_ref[...] = jnp.broadcast_to(v1 / denom, w0_ref.shape)
    w1_ref[...] = jnp.broadcast_to(v2 / denom, w1_ref.shape)

    mask = sel1.astype(jnp.float32) + sel2.astype(jnp.float32)
    counts_f = jnp.sum(mask, axis=0, keepdims=True)
    counts_ref[...] = counts_f.astype(jnp.int32)
    prob_sum = jnp.sum(probs, axis=0, keepdims=True)
    loss_ref[...] = E * jnp.sum((counts_f / T) * (prob_sum / T),
                                axis=1, keepdims=True)

    trows = lax.broadcasted_iota(jnp.int32, (T, T), 0)
    tcols = lax.broadcasted_iota(jnp.int32, (T, T), 1)
    tri = (tcols < trows).astype(jnp.float32)
    excl_cum = lax.dot_general(tri, mask, (((1,), (0,)), ((), ())),
                               preferred_element_type=jnp.float32)
    erows = lax.broadcasted_iota(jnp.int32, (E, E), 0)
    ecols = lax.broadcasted_iota(jnp.int32, (E, E), 1)
    tri_e = (erows < ecols).astype(jnp.float32)
    mask_lt = lax.dot_general(mask, tri_e, (((1,), (0,)), ((), ())),
                              preferred_element_type=jnp.float32)
    offs = jnp.sum(mask_lt, axis=0, keepdims=True)
    slot = offs + excl_cum
    p0_ref[...] = jnp.sum(jnp.where(sel1, slot, 0.0), axis=1,
                          keepdims=True).astype(jnp.int32)
    p1_ref[...] = jnp.sum(jnp.where(sel2, slot, 0.0), axis=1,
                          keepdims=True).astype(jnp.int32)


def _dispatch_body(p0_hbm, p1_hbm, x_hbm, xs_hbm, p0_v, p1_v, rows_v, sem):
    wid = lax.axis_index("s") * _NC + lax.axis_index("c")
    tpw = x_hbm.shape[0] // _NW
    pltpu.sync_copy(p0_hbm.at[wid], p0_v)
    pltpu.sync_copy(p1_hbm.at[wid], p1_v)
    pltpu.sync_copy(x_hbm.at[pl.ds(wid * tpw, tpw)], rows_v)
    c0 = pltpu.async_copy(rows_v, xs_hbm.at[p0_v], sem)
    c1 = pltpu.async_copy(rows_v, xs_hbm.at[p1_v], sem)
    c0.wait()
    c1.wait()


def _ffn_body(sb_ref, se_ref, slo_ref, shi_ref, sf_ref,
              xs_ref, wg_ref, wu_ref, wd_ref, yb_ref):
    s = pl.program_id(0)
    lo = slo_ref[s]
    hi = shi_ref[s]
    rows = lax.broadcasted_iota(jnp.int32, (_BLK, 1), 0)
    msk = jnp.logical_and(rows >= lo, rows < hi)
    x = jnp.where(msk, xs_ref[...], 0.0)
    gate = _silu(lax.dot_general(x, wg_ref[0], (((1,), (1,)), ((), ())),
                                 preferred_element_type=jnp.float32))
    up = lax.dot_general(x, wu_ref[0], (((1,), (1,)), ((), ())),
                         preferred_element_type=jnp.float32)
    contrib = lax.dot_general(gate * up, wd_ref[0], (((1,), (1,)), ((), ())),
                              preferred_element_type=jnp.float32)
    first = sf_ref[s] == 1

    @pl.when(first)
    def _init():
        yb_ref[...] = contrib

    @pl.when(jnp.logical_not(first))
    def _acc():
        yb_ref[...] = yb_ref[...] + contrib


def _combine_body(p0_hbm, p1_hbm, w0_hbm, w1_hbm, yb_hbm, out_hbm,
                  p0_v, p1_v, w0_v, w1_v, y0_v, y1_v, out_v, sem):
    wid = lax.axis_index("s") * _NC + lax.axis_index("c")
    T, H = out_hbm.shape
    tpw = T // _NW
    ht = tpw // 2
    nch = H // 16
    pltpu.sync_copy(p0_hbm.at[wid], p0_v)
    pltpu.sync_copy(p1_hbm.at[wid], p1_v)
    pltpu.sync_copy(w0_hbm.at[pl.ds(wid * tpw, tpw)], w0_v)
    pltpu.sync_copy(w1_hbm.at[pl.ds(wid * tpw, tpw)], w1_v)
    for half in range(2):
        g0 = pltpu.async_copy(yb_hbm.at[p0_v.at[pl.ds(half * ht, ht)]],
                              y0_v, sem)
        g1 = pltpu.async_copy(yb_hbm.at[p1_v.at[pl.ds(half * ht, ht)]],
                              y1_v, sem)
        g0.wait()
        g1.wait()

        def tstep(t, carry):
            w0v = w0_v[half * ht + t, :]
            w1v = w1_v[half * ht + t, :]
            for c in range(nch):
                sl = pl.ds(c * 16, 16)
                out_v[t, sl] = w0v * y0_v[t, sl] + w1v * y1_v[t, sl]
            return carry

        lax.fori_loop(0, ht, tstep, 0)
        pltpu.sync_copy(out_v, out_hbm.at[pl.ds(wid * tpw + half * ht, ht)])


def kernel(x, Wgate, Wg, Wu, Wd):
    b, s_, h = x.shape
    T = b * s_
    E, I, H = Wg.shape
    P = T * _K
    NB = P // _BLK
    NS = NB + E - 1
    x_flat = x.reshape(T, h)

    p0, p1, w0, w1, counts2, loss = pl.pallas_call(
        _router_body,
        out_shape=[
            jax.ShapeDtypeStruct((T, 1), jnp.int32),
            jax.ShapeDtypeStruct((T, 1), jnp.int32),
            jax.ShapeDtypeStruct((T, 16), jnp.float32),
            jax.ShapeDtypeStruct((T, 16), jnp.float32),
            jax.ShapeDtypeStruct((1, E), jnp.int32),
            jax.ShapeDtypeStruct((1, 1), jnp.float32),
        ],
    )(x_flat, Wgate)

    counts = counts2[0]
    offs = jnp.concatenate([jnp.zeros((1,), jnp.int32),
                            jnp.cumsum(counts).astype(jnp.int32)])
    bstart = jnp.arange(NB, dtype=jnp.int32)[:, None] * _BLK
    lo = jnp.maximum(bstart, offs[None, :E])
    hi = jnp.minimum(bstart + _BLK, offs[None, 1:])
    valid = (lo < hi).reshape(-1)
    order = jnp.argsort(jnp.logical_not(valid), stable=True).astype(jnp.int32)
    nvalid = jnp.sum(valid.astype(jnp.int32))
    take = order[jnp.minimum(jnp.arange(NS, dtype=jnp.int32), nvalid - 1)]
    sb = take // E
    se = take % E
    slo = lo.reshape(-1)[take] - sb * _BLK
    shi = hi.reshape(-1)[take] - sb * _BLK
    empty = jnp.arange(NS, dtype=jnp.int32) >= nvalid
    slo = jnp.where(empty, shi, slo)
    sfirst = jnp.concatenate([jnp.ones((1,), jnp.int32),
                              (sb[1:] != sb[:-1]).astype(jnp.int32)])

    tpw = T // _NW
    p0w = p0.reshape(_NW, tpw)
    p1w = p1.reshape(_NW, tpw)

    mesh = plsc.VectorSubcoreMesh(core_axis_name="c", subcore_axis_name="s")
    xs = pl.kernel(
        _dispatch_body,
        out_type=jax.ShapeDtypeStruct((P, H), jnp.float32),
        mesh=mesh,
        scratch_types=[
            pltpu.VMEM((tpw,), jnp.int32),
            pltpu.VMEM((tpw,), jnp.int32),
            pltpu.VMEM((tpw, H), jnp.float32),
            pltpu.SemaphoreType.DMA,
        ],
        compiler_params=pltpu.CompilerParams(needs_layout_passes=False),
    )(p0w, p1w, x_flat)

    yb = pl.pallas_call(
        _ffn_body,
        grid_spec=pltpu.PrefetchScalarGridSpec(
            num_scalar_prefetch=5,
            grid=(NS,),
            in_specs=[
                pl.BlockSpec((_BLK, H),
                             lambda s, sb, se, slo, shi, sf: (sb[s], 0)),
                pl.BlockSpec((1, I, H),
                             lambda s, sb, se, slo, shi, sf: (se[s], 0, 0)),
                pl.BlockSpec((1, I, H),
                             lambda s, sb, se, slo, shi, sf: (se[s], 0, 0)),
                pl.BlockSpec((1, H, I),
                             lambda s, sb, se, slo, shi, sf: (se[s], 0, 0)),
            ],
            out_specs=pl.BlockSpec((_BLK, H),
                                   lambda s, sb, se, slo, shi, sf: (sb[s], 0)),
        ),
        out_shape=jax.ShapeDtypeStruct((P, H), jnp.float32),
        compiler_params=pltpu.CompilerParams(
            dimension_semantics=("arbitrary",),
        ),
    )(sb, se, slo, shi, sfirst, xs, Wg, Wu, Wd)

    out = pl.kernel(
        _combine_body,
        out_type=jax.ShapeDtypeStruct((T, H), jnp.float32),
        mesh=mesh,
        scratch_types=[
            pltpu.VMEM((tpw,), jnp.int32),
            pltpu.VMEM((tpw,), jnp.int32),
            pltpu.VMEM((tpw, 16), jnp.float32),
            pltpu.VMEM((tpw, 16), jnp.float32),
            pltpu.VMEM((tpw // 2, H), jnp.float32),
            pltpu.VMEM((tpw // 2, H), jnp.float32),
            pltpu.VMEM((tpw // 2, H), jnp.float32),
            pltpu.SemaphoreType.DMA,
        ],
        compiler_params=pltpu.CompilerParams(needs_layout_passes=False),
    )(p0w, p1w, w0, w1, yb)

    return out.reshape(b, s_, h), loss.reshape(())


# speedup vs baseline: 1.2063x; 1.2063x over previous
import jax
import jax.numpy as jnp
from jax import lax
from jax.experimental import pallas as pl
from jax.experimental.pallas import tpu as pltpu
from jax.experimental.pallas import tpu_sc as plsc

_NC = 2
_NSC = 16
_NW = _NC * _NSC

_K = 2
_BLK = 512


def _silu(v):
    return v / (1.0 + jnp.exp(-v))


def _router_body(x_ref, wgate_ref, p0_ref, p1_ref, w0_ref, w1_ref,
                 counts_ref, loss_ref):
    x = x_ref[...]
    wg = wgate_ref[...]
    T = x.shape[0]
    E = wg.shape[0]
    logits = lax.dot_general(x, wg, (((1,), (1,)), ((), ())),
                             preferred_element_type=jnp.float32)
    m = jnp.max(logits, axis=-1, keepdims=True)
    ex = jnp.exp(logits - m)
    probs = ex / jnp.sum(ex, axis=-1, keepdims=True)
    eids = lax.broadcasted_iota(jnp.int32, probs.shape, 1)
    v1 = jnp.max(probs, axis=-1, keepdims=True)
    i1 = jnp.min(jnp.where(probs == v1, eids, E), axis=-1, keepdims=True)
    sel1 = eids == i1
    p2 = jnp.where(sel1, -1.0, probs)
    v2 = jnp.max(p2, axis=-1, keepdims=True)
    i2 = jnp.min(jnp.where(p2 == v2, eids, E), axis=-1, keepdims=True)
    sel2 = eids == i2
    denom = v1 + v2
    w0_ref[...] = jnp.broadcast_to(v1 / denom, w0_ref.shape)
    w1_ref[...] = jnp.broadcast_to(v2 / denom, w1_ref.shape)

    mask = sel1.astype(jnp.float32) + sel2.astype(jnp.float32)
    counts_f = jnp.sum(mask, axis=0, keepdims=True)
    counts_ref[...] = counts_f.astype(jnp.int32)
    prob_sum = jnp.sum(probs, axis=0, keepdims=True)
    loss_ref[...] = E * jnp.sum((counts_f / T) * (prob_sum / T),
                                axis=1, keepdims=True)

    trows = lax.broadcasted_iota(jnp.int32, (T, T), 0)
    tcols = lax.broadcasted_iota(jnp.int32, (T, T), 1)
    tri = (tcols < trows).astype(jnp.float32)
    excl_cum = lax.dot_general(tri, mask, (((1,), (0,)), ((), ())),
                               preferred_element_type=jnp.float32)
    erows = lax.broadcasted_iota(jnp.int32, (E, E), 0)
    ecols = lax.broadcasted_iota(jnp.int32, (E, E), 1)
    tri_e = (erows < ecols).astype(jnp.float32)
    mask_lt = lax.dot_general(mask, tri_e, (((1,), (0,)), ((), ())),
                              preferred_element_type=jnp.float32)
    offs = jnp.sum(mask_lt, axis=0, keepdims=True)
    slot = offs + excl_cum
    p0_ref[...] = jnp.sum(jnp.where(sel1, slot, 0.0), axis=1,
                          keepdims=True).astype(jnp.int32)
    p1_ref[...] = jnp.sum(jnp.where(sel2, slot, 0.0), axis=1,
                          keepdims=True).astype(jnp.int32)


def _dispatch_body(p0_hbm, p1_hbm, x_hbm, xs_hbm, p0_v, p1_v, rows_v, sem):
    wid = lax.axis_index("s") * _NC + lax.axis_index("c")
    tpw = x_hbm.shape[0] // _NW
    pltpu.sync_copy(p0_hbm.at[wid], p0_v)
    pltpu.sync_copy(p1_hbm.at[wid], p1_v)
    pltpu.sync_copy(x_hbm.at[pl.ds(wid * tpw, tpw)], rows_v)
    c0 = pltpu.async_copy(rows_v, xs_hbm.at[p0_v], sem)
    c1 = pltpu.async_copy(rows_v, xs_hbm.at[p1_v], sem)
    c0.wait()
    c1.wait()


def _ffn_body(sb_ref, se_ref, slo_ref, shi_ref, sf_ref,
              xs_ref, wg_ref, wu_ref, wd_ref, yb_ref):
    s = pl.program_id(0)
    lo = slo_ref[s]
    hi = shi_ref[s]
    rows = lax.broadcasted_iota(jnp.int32, (_BLK, 1), 0)
    msk = jnp.logical_and(rows >= lo, rows < hi)
    x = jnp.where(msk, xs_ref[...], 0.0)
    gate = _silu(lax.dot_general(x, wg_ref[0], (((1,), (1,)), ((), ())),
                                 preferred_element_type=jnp.float32))
    up = lax.dot_general(x, wu_ref[0], (((1,), (1,)), ((), ())),
                         preferred_element_type=jnp.float32)
    contrib = lax.dot_general(gate * up, wd_ref[0], (((1,), (1,)), ((), ())),
                              preferred_element_type=jnp.float32)
    first = sf_ref[s] == 1

    @pl.when(first)
    def _init():
        yb_ref[...] = contrib

    @pl.when(jnp.logical_not(first))
    def _acc():
        yb_ref[...] = yb_ref[...] + contrib


def _combine_body(p0_hbm, p1_hbm, w0_hbm, w1_hbm, yb_hbm, out_hbm,
                  p0_v, p1_v, w0_v, w1_v, y0_v, y1_v, out_v, sem):
    wid = lax.axis_index("s") * _NC + lax.axis_index("c")
    T, H = out_hbm.shape
    tpw = T // _NW
    ht = tpw // 2
    nch = H // 16
    pltpu.sync_copy(p0_hbm.at[wid], p0_v)
    pltpu.sync_copy(p1_hbm.at[wid], p1_v)
    pltpu.sync_copy(w0_hbm.at[pl.ds(wid * tpw, tpw)], w0_v)
    pltpu.sync_copy(w1_hbm.at[pl.ds(wid * tpw, tpw)], w1_v)
    for half in range(2):
        g0 = pltpu.async_copy(yb_hbm.at[p0_v.at[pl.ds(half * ht, ht)]],
                              y0_v, sem)
        g1 = pltpu.async_copy(yb_hbm.at[p1_v.at[pl.ds(half * ht, ht)]],
                              y1_v, sem)
        g0.wait()
        g1.wait()

        def tstep(t, carry):
            w0v = w0_v[half * ht + t, :]
            w1v = w1_v[half * ht + t, :]
            for c in range(nch):
                sl = pl.ds(c * 16, 16)
                out_v[t, sl] = w0v * y0_v[t, sl] + w1v * y1_v[t, sl]
            return carry

        lax.fori_loop(0, ht, tstep, 0)
        pltpu.sync_copy(out_v, out_hbm.at[pl.ds(wid * tpw + half * ht, ht)])


def kernel(x, Wgate, Wg, Wu, Wd):
    b, s_, h = x.shape
    T = b * s_
    E, I, H = Wg.shape
    P = T * _K
    NB = P // _BLK
    NS = NB + E - 1
    x_flat = x.reshape(T, h)

    p0, p1, w0, w1, counts2, loss = pl.pallas_call(
        _router_body,
        out_shape=[
            jax.ShapeDtypeStruct((T, 1), jnp.int32),
            jax.ShapeDtypeStruct((T, 1), jnp.int32),
            jax.ShapeDtypeStruct((T, 16), jnp.float32),
            jax.ShapeDtypeStruct((T, 16), jnp.float32),
            jax.ShapeDtypeStruct((1, E), jnp.int32),
            jax.ShapeDtypeStruct((1, 1), jnp.float32),
        ],
    )(x_flat, Wgate)

    counts = counts2[0]
    offs = jnp.concatenate([jnp.zeros((1,), jnp.int32),
                            jnp.cumsum(counts).astype(jnp.int32)])
    bstart = jnp.arange(NB, dtype=jnp.int32)[:, None] * _BLK
    lo = jnp.maximum(bstart, offs[None, :E])
    hi = jnp.minimum(bstart + _BLK, offs[None, 1:])
    valid = (lo < hi).reshape(-1)
    order = jnp.argsort(jnp.logical_not(valid), stable=True).astype(jnp.int32)
    nvalid = jnp.sum(valid.astype(jnp.int32))
    take = order[jnp.minimum(jnp.arange(NS, dtype=jnp.int32), nvalid - 1)]
    sb = take // E
    se = take % E
    slo = lo.reshape(-1)[take] - sb * _BLK
    shi = hi.reshape(-1)[take] - sb * _BLK
    empty = jnp.arange(NS, dtype=jnp.int32) >= nvalid
    slo = jnp.where(empty, shi, slo)
    sfirst = jnp.concatenate([jnp.ones((1,), jnp.int32),
                              (sb[1:] != sb[:-1]).astype(jnp.int32)])

    tpw = T // _NW
    p0w = p0.reshape(_NW, tpw)
    p1w = p1.reshape(_NW, tpw)

    mesh = plsc.VectorSubcoreMesh(core_axis_name="c", subcore_axis_name="s")
    xs = pl.kernel(
        _dispatch_body,
        out_type=jax.ShapeDtypeStruct((P, H), jnp.float32),
        mesh=mesh,
        scratch_types=[
            pltpu.VMEM((tpw,), jnp.int32),
            pltpu.VMEM((tpw,), jnp.int32),
            pltpu.VMEM((tpw, H), jnp.float32),
            pltpu.SemaphoreType.DMA,
        ],
        compiler_params=pltpu.CompilerParams(needs_layout_passes=False),
    )(p0w, p1w, x_flat)

    yb = pl.pallas_call(
        _ffn_body,
        grid_spec=pltpu.PrefetchScalarGridSpec(
            num_scalar_prefetch=5,
            grid=(NS,),
            in_specs=[
                pl.BlockSpec((_BLK, H),
                             lambda s, sb, se, slo, shi, sf: (sb[s], 0)),
                pl.BlockSpec((1, I, H),
                             lambda s, sb, se, slo, shi, sf: (se[s], 0, 0)),
                pl.BlockSpec((1, I, H),
                             lambda s, sb, se, slo, shi, sf: (se[s], 0, 0)),
                pl.BlockSpec((1, H, I),
                             lambda s, sb, se, slo, shi, sf: (se[s], 0, 0)),
            ],
            out_specs=pl.BlockSpec((_BLK, H),
                                   lambda s, sb, se, slo, shi, sf: (sb[s], 0)),
        ),
        out_shape=jax.ShapeDtypeStruct((P, H), jnp.float32),
        compiler_params=pltpu.CompilerParams(
            dimension_semantics=("arbitrary",),
        ),
    )(sb, se, slo, shi, sfirst, xs, Wg, Wu, Wd)

    out = pl.kernel(
        _combine_body,
        out_type=jax.ShapeDtypeStruct((T, H), jnp.float32),
        mesh=mesh,
        scratch_types=[
            pltpu.VMEM((tpw,), jnp.int32),
            pltpu.VMEM((tpw,), jnp.int32),
            pltpu.VMEM((tpw, 16), jnp.float32),
            pltpu.VMEM((tpw, 16), jnp.float32),
            pltpu.VMEM((tpw // 2, H), jnp.float32),
            pltpu.VMEM((tpw // 2, H), jnp.float32),
            pltpu.VMEM((tpw // 2, H), jnp.float32),
            pltpu.SemaphoreType.DMA,
        ],
        compiler_params=pltpu.CompilerParams(needs_layout_passes=False),
    )(p0w, p1w, w0, w1, yb)

    return out.reshape(b, s_, h), loss.reshape(())


# speedup vs baseline: 1.2195x; 1.0109x over previous
import jax
import jax.numpy as jnp
from jax import lax
from jax.experimental import pallas as pl
from jax.experimental.pallas import tpu as pltpu
from jax.experimental.pallas import tpu_sc as plsc

_NC = 2
_NSC = 16
_NW = _NC * _NSC

_K = 2
_BLK = 512


def _silu(v):
    return v / (1.0 + jnp.exp(-v))


def _router_body(x_ref, wgate_ref, p0_ref, p1_ref, w0_ref, w1_ref,
                 counts_ref, loss_ref):
    x = x_ref[...]
    wg = wgate_ref[...]
    T = x.shape[0]
    E = wg.shape[0]
    logits = lax.dot_general(x, wg, (((1,), (1,)), ((), ())),
                             preferred_element_type=jnp.float32)
    m = jnp.max(logits, axis=-1, keepdims=True)
    ex = jnp.exp(logits - m)
    probs = ex / jnp.sum(ex, axis=-1, keepdims=True)
    eids = lax.broadcasted_iota(jnp.int32, probs.shape, 1)
    v1 = jnp.max(probs, axis=-1, keepdims=True)
    i1 = jnp.min(jnp.where(probs == v1, eids, E), axis=-1, keepdims=True)
    sel1 = eids == i1
    p2 = jnp.where(sel1, -1.0, probs)
    v2 = jnp.max(p2, axis=-1, keepdims=True)
    i2 = jnp.min(jnp.where(p2 == v2, eids, E), axis=-1, keepdims=True)
    sel2 = eids == i2
    denom = v1 + v2
    w0_ref[...] = jnp.broadcast_to(v1 / denom, w0_ref.shape)
    w1_ref[...] = jnp.broadcast_to(v2 / denom, w1_ref.shape)

    mask = sel1.astype(jnp.float32) + sel2.astype(jnp.float32)
    counts_f = jnp.sum(mask, axis=0, keepdims=True)
    counts_ref[...] = counts_f.astype(jnp.int32)
    prob_sum = jnp.sum(probs, axis=0, keepdims=True)
    loss_ref[...] = E * jnp.sum((counts_f / T) * (prob_sum / T),
                                axis=1, keepdims=True)

    trows = lax.broadcasted_iota(jnp.int32, (T, T), 0)
    tcols = lax.broadcasted_iota(jnp.int32, (T, T), 1)
    tri = (tcols < trows).astype(jnp.float32)
    excl_cum = lax.dot_general(tri, mask, (((1,), (0,)), ((), ())),
                               preferred_element_type=jnp.float32)
    erows = lax.broadcasted_iota(jnp.int32, (E, E), 0)
    ecols = lax.broadcasted_iota(jnp.int32, (E, E), 1)
    tri_e = (erows < ecols).astype(jnp.float32)
    mask_lt = lax.dot_general(mask, tri_e, (((1,), (0,)), ((), ())),
                              preferred_element_type=jnp.float32)
    offs = jnp.sum(mask_lt, axis=0, keepdims=True)
    slot = offs + excl_cum
    p0_ref[...] = jnp.sum(jnp.where(sel1, slot, 0.0), axis=1,
                          keepdims=True).astype(jnp.int32)
    p1_ref[...] = jnp.sum(jnp.where(sel2, slot, 0.0), axis=1,
                          keepdims=True).astype(jnp.int32)


def _dispatch_body(p0_hbm, p1_hbm, x_hbm, xs_hbm, p0_v, p1_v, rows_v, sem):
    wid = lax.axis_index("s") * _NC + lax.axis_index("c")
    tpw = x_hbm.shape[0] // _NW
    pltpu.sync_copy(p0_hbm.at[wid], p0_v)
    pltpu.sync_copy(p1_hbm.at[wid], p1_v)
    pltpu.sync_copy(x_hbm.at[pl.ds(wid * tpw, tpw)], rows_v)
    c0 = pltpu.async_copy(rows_v, xs_hbm.at[p0_v], sem)
    c1 = pltpu.async_copy(rows_v, xs_hbm.at[p1_v], sem)
    c0.wait()
    c1.wait()


def _ffn_body(sb_ref, se_ref, slo_ref, shi_ref, sf_ref,
              xs_ref, wg_ref, wu_ref, wd_ref, yb_ref):
    s = pl.program_id(0)
    lo = slo_ref[s]
    hi = shi_ref[s]
    rows = lax.broadcasted_iota(jnp.int32, (_BLK, 1), 0)
    msk = jnp.logical_and(rows >= lo, rows < hi)
    x = jnp.where(msk, xs_ref[...], 0.0)
    gate = _silu(lax.dot_general(x, wg_ref[0], (((1,), (1,)), ((), ())),
                                 preferred_element_type=jnp.float32))
    up = lax.dot_general(x, wu_ref[0], (((1,), (1,)), ((), ())),
                         preferred_element_type=jnp.float32)
    contrib = lax.dot_general(gate * up, wd_ref[0], (((1,), (1,)), ((), ())),
                              preferred_element_type=jnp.float32)
    first = sf_ref[s] == 1

    @pl.when(first)
    def _init():
        yb_ref[...] = contrib

    @pl.when(jnp.logical_not(first))
    def _acc():
        yb_ref[...] = yb_ref[...] + contrib


def _combine_body(p0_hbm, p1_hbm, w0_hbm, w1_hbm, yb_hbm, out_hbm,
                  p0_v, p1_v, w0_v, w1_v, y0a_v, y1a_v, y0b_v, y1b_v,
                  out_v, sem_a, sem_b):
    wid = lax.axis_index("s") * _NC + lax.axis_index("c")
    T, H = out_hbm.shape
    tpw = T // _NW
    ht = tpw // 2
    nch = H // 16
    pltpu.sync_copy(p0_hbm.at[wid], p0_v)
    pltpu.sync_copy(p1_hbm.at[wid], p1_v)
    pltpu.sync_copy(w0_hbm.at[pl.ds(wid * tpw, tpw)], w0_v)
    pltpu.sync_copy(w1_hbm.at[pl.ds(wid * tpw, tpw)], w1_v)
    qt = tpw // 4
    nq = 4
    bufs = [(y0a_v, y1a_v), (y0b_v, y1b_v)]
    sems = [sem_a, sem_b]

    def issue(ci):
        y0_v, y1_v = bufs[ci % 2]
        c0 = pltpu.async_copy(yb_hbm.at[p0_v.at[pl.ds(ci * qt, qt)]],
                              y0_v, sems[ci % 2])
        c1 = pltpu.async_copy(yb_hbm.at[p1_v.at[pl.ds(ci * qt, qt)]],
                              y1_v, sems[ci % 2])
        return c0, c1

    pend = issue(0)
    for ci in range(nq):
        nxt = issue(ci + 1) if ci + 1 < nq else None
        pend[0].wait()
        pend[1].wait()
        y0_v, y1_v = bufs[ci % 2]

        def tstep(t, carry):
            w0v = w0_v[ci * qt + t, :]
            w1v = w1_v[ci * qt + t, :]
            for c in range(nch):
                sl = pl.ds(c * 16, 16)
                out_v[t, sl] = w0v * y0_v[t, sl] + w1v * y1_v[t, sl]
            return carry

        lax.fori_loop(0, qt, tstep, 0)
        pltpu.sync_copy(out_v, out_hbm.at[pl.ds(wid * tpw + ci * qt, qt)])
        pend = nxt


def kernel(x, Wgate, Wg, Wu, Wd):
    b, s_, h = x.shape
    T = b * s_
    E, I, H = Wg.shape
    P = T * _K
    NB = P // _BLK
    NS = NB + E - 1
    x_flat = x.reshape(T, h)

    p0, p1, w0, w1, counts2, loss = pl.pallas_call(
        _router_body,
        out_shape=[
            jax.ShapeDtypeStruct((T, 1), jnp.int32),
            jax.ShapeDtypeStruct((T, 1), jnp.int32),
            jax.ShapeDtypeStruct((T, 16), jnp.float32),
            jax.ShapeDtypeStruct((T, 16), jnp.float32),
            jax.ShapeDtypeStruct((1, E), jnp.int32),
            jax.ShapeDtypeStruct((1, 1), jnp.float32),
        ],
    )(x_flat, Wgate)

    counts = counts2[0]
    offs = jnp.concatenate([jnp.zeros((1,), jnp.int32),
                            jnp.cumsum(counts).astype(jnp.int32)])
    bstart = jnp.arange(NB, dtype=jnp.int32)[:, None] * _BLK
    lo = jnp.maximum(bstart, offs[None, :E])
    hi = jnp.minimum(bstart + _BLK, offs[None, 1:])
    valid = (lo < hi).reshape(-1)
    order = jnp.argsort(jnp.logical_not(valid), stable=True).astype(jnp.int32)
    nvalid = jnp.sum(valid.astype(jnp.int32))
    take = order[jnp.minimum(jnp.arange(NS, dtype=jnp.int32), nvalid - 1)]
    sb = take // E
    se = take % E
    slo = lo.reshape(-1)[take] - sb * _BLK
    shi = hi.reshape(-1)[take] - sb * _BLK
    empty = jnp.arange(NS, dtype=jnp.int32) >= nvalid
    slo = jnp.where(empty, shi, slo)
    sfirst = jnp.concatenate([jnp.ones((1,), jnp.int32),
                              (sb[1:] != sb[:-1]).astype(jnp.int32)])

    tpw = T // _NW
    p0w = p0.reshape(_NW, tpw)
    p1w = p1.reshape(_NW, tpw)

    mesh = plsc.VectorSubcoreMesh(core_axis_name="c", subcore_axis_name="s")
    xs = pl.kernel(
        _dispatch_body,
        out_type=jax.ShapeDtypeStruct((P, H), jnp.float32),
        mesh=mesh,
        scratch_types=[
            pltpu.VMEM((tpw,), jnp.int32),
            pltpu.VMEM((tpw,), jnp.int32),
            pltpu.VMEM((tpw, H), jnp.float32),
            pltpu.SemaphoreType.DMA,
        ],
        compiler_params=pltpu.CompilerParams(needs_layout_passes=False),
    )(p0w, p1w, x_flat)

    yb = pl.pallas_call(
        _ffn_body,
        grid_spec=pltpu.PrefetchScalarGridSpec(
            num_scalar_prefetch=5,
            grid=(NS,),
            in_specs=[
                pl.BlockSpec((_BLK, H),
                             lambda s, sb, se, slo, shi, sf: (sb[s], 0)),
                pl.BlockSpec((1, I, H),
                             lambda s, sb, se, slo, shi, sf: (se[s], 0, 0)),
                pl.BlockSpec((1, I, H),
                             lambda s, sb, se, slo, shi, sf: (se[s], 0, 0)),
                pl.BlockSpec((1, H, I),
                             lambda s, sb, se, slo, shi, sf: (se[s], 0, 0)),
            ],
            out_specs=pl.BlockSpec((_BLK, H),
                                   lambda s, sb, se, slo, shi, sf: (sb[s], 0)),
        ),
        out_shape=jax.ShapeDtypeStruct((P, H), jnp.float32),
        compiler_params=pltpu.CompilerParams(
            dimension_semantics=("arbitrary",),
        ),
    )(sb, se, slo, shi, sfirst, xs, Wg, Wu, Wd)

    out = pl.kernel(
        _combine_body,
        out_type=jax.ShapeDtypeStruct((T, H), jnp.float32),
        mesh=mesh,
        scratch_types=[
            pltpu.VMEM((tpw,), jnp.int32),
            pltpu.VMEM((tpw,), jnp.int32),
            pltpu.VMEM((tpw, 16), jnp.float32),
            pltpu.VMEM((tpw, 16), jnp.float32),
            pltpu.VMEM((tpw // 4, H), jnp.float32),
            pltpu.VMEM((tpw // 4, H), jnp.float32),
            pltpu.VMEM((tpw // 4, H), jnp.float32),
            pltpu.VMEM((tpw // 4, H), jnp.float32),
            pltpu.VMEM((tpw // 4, H), jnp.float32),
            pltpu.SemaphoreType.DMA,
            pltpu.SemaphoreType.DMA,
        ],
        compiler_params=pltpu.CompilerParams(needs_layout_passes=False),
    )(p0w, p1w, w0, w1, yb)

    return out.reshape(b, s_, h), loss.reshape(())
